# Initial kernel scaffold; baseline (speedup 1.0000x reference)
import jax
import jax.numpy as jnp
from jax.experimental import pallas as pl


def kernel(x, edge_index, W1, b1, W2, b2, W3, b3, W4, b4):
    raise NotImplementedError("write your pallas kernel here")


# speedup vs baseline: 38.1243x; 38.1243x over previous
import functools

import jax
import jax.numpy as jnp
from jax import lax
from jax.experimental import pallas as pl
from jax.experimental.pallas import tpu as pltpu
from jax.experimental.pallas import tpu_sc as plsc

N = 50000
E = 1600000

NC = 2
NS = 16
NW = NC * NS

NP = 51200
STRIPE = NP // NS

EPAD = 1638400
ER = EPAD // 128
ER_W = ER // NW
ER_T = ER // NS
G = 8
F32 = jnp.float32

_MESH = plsc.VectorSubcoreMesh(core_axis_name="c", subcore_axis_name="s")


def _stage_rows(hbm_slice, sp_ref, buf, r0, rows, chunk):
    for j in range(rows // chunk):
        r = r0 + j * chunk
        pltpu.sync_copy(hbm_slice.at[pl.ds(r, chunk)], buf)
        pltpu.sync_copy(buf, sp_ref.at[pl.ds(r, chunk)])


def _dump_rows(sp_ref, hbm_slice, buf, r0, rows, chunk):
    for j in range(rows // chunk):
        r = r0 + j * chunk
        pltpu.sync_copy(sp_ref.at[pl.ds(r, chunk)], buf)
        pltpu.sync_copy(buf, hbm_slice.at[pl.ds(r, chunk)])


@functools.partial(
    pl.kernel, mesh=_MESH,
    compiler_params=pltpu.CompilerParams(use_tc_tiling_on_sc=False),
    out_type=jax.ShapeDtypeStruct((NC, NP), F32),
    scratch_types=[
        pltpu.VMEM((G, 128), jnp.int32),
        pltpu.VMEM((G, 128), F32),
        pltpu.VMEM((STRIPE,), F32),
        pltpu.VMEM_SHARED((NP,), F32),
    ])
def _deg_sc(dst_hbm, zeros_hbm, out_hbm, didx, ones_v, stage_v, acc_sp):
    c = lax.axis_index("c")
    s = lax.axis_index("s")
    st = pl.multiple_of(s * STRIPE, 8)
    pltpu.sync_copy(zeros_hbm.at[pl.ds(st, STRIPE)], stage_v)
    pltpu.sync_copy(stage_v, acc_sp.at[pl.ds(st, STRIPE)])
    one = jnp.full((16,), 1.0, F32)
    for j in range(G):
        for i in range(8):
            ones_v[j, pl.ds(i * 16, 16)] = one
    plsc.subcore_barrier()

    row0 = (c * NS + s) * ER_W

    @pl.loop(0, ER_W // G)
    def _(grp):
        rb = pl.multiple_of(row0 + grp * G, G)
        pltpu.sync_copy(dst_hbm.at[pl.ds(rb, G)], didx)
        for j in range(G):
            pltpu.sync_copy(ones_v.at[j], acc_sp.at[didx.at[j]], add=True)

    plsc.subcore_barrier()
    _dump_rows(acc_sp, out_hbm.at[c], stage_v, st, STRIPE, STRIPE)


def _make_prop1():
    @functools.partial(
        pl.kernel, mesh=_MESH,
        compiler_params=pltpu.CompilerParams(use_tc_tiling_on_sc=False),
        out_type=jax.ShapeDtypeStruct((NC, NP), F32),
        scratch_types=[
            pltpu.VMEM((G, 128), jnp.int32),
            pltpu.VMEM((G, 128), jnp.int32),
            pltpu.VMEM((G, 128), F32),
            pltpu.VMEM((STRIPE,), F32),
            pltpu.VMEM_SHARED((NP,), F32),
            pltpu.VMEM_SHARED((NP,), F32),
        ])
    def k(src_hbm, dst_hbm, g_hbm, out_hbm, sidx, didx, rows_v, stage_v,
          g_sp, acc_sp):
        c = lax.axis_index("c")
        s = lax.axis_index("s")
        st = pl.multiple_of(s * STRIPE, 8)
        pltpu.sync_copy(g_hbm.at[pl.ds(st, STRIPE)], stage_v)
        pltpu.sync_copy(stage_v, g_sp.at[pl.ds(st, STRIPE)])
        pltpu.sync_copy(stage_v, acc_sp.at[pl.ds(st, STRIPE)])
        plsc.subcore_barrier()

        row0 = (c * NS + s) * ER_W

        @pl.loop(0, ER_W // G)
        def _(grp):
            rb = pl.multiple_of(row0 + grp * G, G)
            pltpu.sync_copy(src_hbm.at[pl.ds(rb, G)], sidx)
            pltpu.sync_copy(dst_hbm.at[pl.ds(rb, G)], didx)
            for j in range(G):
                pltpu.sync_copy(g_sp.at[sidx.at[j]], rows_v.at[j])
                pltpu.sync_copy(rows_v.at[j], acc_sp.at[didx.at[j]], add=True)

        plsc.subcore_barrier()
        _dump_rows(acc_sp, out_hbm.at[c], stage_v, st, STRIPE, STRIPE)

    return k


def _make_propF(F, feature_split):
    CH = 800

    g_shape = (NC, NP, F) if feature_split else (NP, F)

    @functools.partial(
        pl.kernel, mesh=_MESH,
        compiler_params=pltpu.CompilerParams(use_tc_tiling_on_sc=False),
        out_type=jax.ShapeDtypeStruct((NC, NP, F), F32),
        scratch_types=[
            pltpu.VMEM((G, 128), jnp.int32),
            pltpu.VMEM((G, 128), jnp.int32),
            pltpu.VMEM((G, 128, F), F32),
            pltpu.VMEM((CH, F), F32),
            pltpu.VMEM_SHARED((NP, F), F32),
        ])
    def k(src_hbm, dst_hbm, g_hbm, out_hbm, sidx, didx, rows_v, stage_v,
          acc_sp):
        c = lax.axis_index("c")
        s = lax.axis_index("s")
        st = pl.multiple_of(s * STRIPE, 8)

        g_src = g_hbm.at[c] if feature_split else g_hbm
        for j in range(STRIPE // CH):
            r = st + j * CH
            pltpu.sync_copy(g_src.at[pl.ds(r, CH)], stage_v)
            pltpu.sync_copy(stage_v, acc_sp.at[pl.ds(r, CH)])
        plsc.subcore_barrier()

        if feature_split:
            row0 = s * ER_T
            n_grp = ER_T // G
        else:
            row0 = (c * NS + s) * ER_W
            n_grp = ER_W // G

        @pl.loop(0, n_grp)
        def _(grp):
            rb = pl.multiple_of(row0 + grp * G, G)
            pltpu.sync_copy(src_hbm.at[pl.ds(rb, G)], sidx)
            pltpu.sync_copy(dst_hbm.at[pl.ds(rb, G)], didx)
            for j in range(G):
                pltpu.sync_copy(g_src.at[sidx.at[j]], rows_v.at[j])
                pltpu.sync_copy(rows_v.at[j], acc_sp.at[didx.at[j]], add=True)

        plsc.subcore_barrier()
        _dump_rows(acc_sp, out_hbm.at[c], stage_v, st, STRIPE, CH)

    return k, g_shape


_prop1_sc = _make_prop1()
_prop32_sc, _ = _make_propF(16, feature_split=True)
_prop16_sc, _ = _make_propF(16, feature_split=False)


BN = 10240
GRID = NP // BN


def _b1_body(degp_ref, x_ref, dinv_ref, g1_ref):
    deg = degp_ref[0] + degp_ref[1] + 1.0
    dinv = lax.rsqrt(deg)
    dinv_ref[...] = dinv
    g1_ref[...] = dinv * x_ref[...]


def _b2_body(p1_ref, g1_ref, dinv_ref, w1_ref, b1_ref, w2_ref, out_ref):
    dinv = dinv_ref[...]
    p0 = dinv * (p1_ref[0] + p1_ref[1] - g1_ref[...])
    h1 = jnp.maximum(p0[:, None] * w1_ref[0][None, :] + b1_ref[...], 0.0)
    a2 = jnp.dot(h1, w2_ref[...], preferred_element_type=F32)
    g2 = dinv[:, None] * a2
    out_ref[0] = g2[:, :16]
    out_ref[1] = g2[:, 16:]


def _b3_body(p2_ref, dinv_ref, b2_ref, w3_ref, out_ref):
    dinv = dinv_ref[...]
    acc = jnp.concatenate([p2_ref[0], p2_ref[1]], axis=-1)
    h2 = jnp.maximum(dinv[:, None] * acc + b2_ref[...], 0.0)
    a3 = jnp.dot(h2, w3_ref[...], preferred_element_type=F32)
    out_ref[...] = dinv[:, None] * a3


def _b4_body(p3_ref, g3_ref, dinv_ref, b3_ref, w4_ref, out_ref):
    dinv = dinv_ref[...]
    acc = p3_ref[0] + p3_ref[1] - g3_ref[...]
    h3 = jnp.maximum(dinv[:, None] * acc + b3_ref[...], 0.0)
    a4 = jnp.sum(h3 * w4_ref[0][None, :], axis=-1)
    out_ref[...] = dinv * a4


def _b5_body(p4_ref, g4_ref, dinv_ref, b4_ref, out_ref):
    out_ref[...] = (dinv_ref[...] * (p4_ref[0] + p4_ref[1] - g4_ref[...])
                    + b4_ref[0, 0])


def _vec_spec():
    return pl.BlockSpec((BN,), lambda i: (i,))


def _pair_spec():
    return pl.BlockSpec((2, BN), lambda i: (0, i))


def _mat_spec(F):
    return pl.BlockSpec((BN, F), lambda i: (i, 0))


def _pairmat_spec(F):
    return pl.BlockSpec((2, BN, F), lambda i: (0, i, 0))


def _full(shape):
    return pl.BlockSpec(shape, lambda i: tuple(0 for _ in shape))


def kernel(x, edge_index, W1, b1, W2, b2, W3, b3, W4, b4):
    src = edge_index[0].astype(jnp.int32)
    dst = edge_index[1].astype(jnp.int32)

    pad_e = EPAD - E
    pad_ids = (N + (jnp.arange(pad_e, dtype=jnp.int32) % (NP - N)))
    src2d = jnp.concatenate([src, pad_ids]).reshape(ER, 128)
    dst2d = jnp.concatenate([dst, pad_ids]).reshape(ER, 128)

    xp = jnp.pad(x[:, 0], (0, NP - N))
    zeros_n = jnp.zeros((NP,), F32)
    b1r = b1.reshape(1, 64)
    b2r = b2.reshape(1, 32)
    b3r = b3.reshape(1, 16)
    b4r = b4.reshape(1, 1)

    degp = _deg_sc(dst2d, zeros_n)

    dinv, g1 = pl.pallas_call(
        _b1_body,
        grid=(GRID,),
        in_specs=[_pair_spec(), _vec_spec()],
        out_specs=[_vec_spec(), _vec_spec()],
        out_shape=[jax.ShapeDtypeStruct((NP,), F32),
                   jax.ShapeDtypeStruct((NP,), F32)],
    )(degp, xp)

    p1 = _prop1_sc(src2d, dst2d, g1)

    g2 = pl.pallas_call(
        _b2_body,
        grid=(GRID,),
        in_specs=[_pair_spec(), _vec_spec(), _vec_spec(),
                  _full((1, 64)), _full((1, 64)), _full((64, 32))],
        out_specs=_pairmat_spec(16),
        out_shape=jax.ShapeDtypeStruct((2, NP, 16), F32),
    )(p1, g1, dinv, W1, b1r, W2)

    p2 = _prop32_sc(src2d, dst2d, g2)

    g3 = pl.pallas_call(
        _b3_body,
        grid=(GRID,),
        in_specs=[_pairmat_spec(16), _vec_spec(),
                  _full((1, 32)), _full((32, 16))],
        out_specs=_mat_spec(16),
        out_shape=jax.ShapeDtypeStruct((NP, 16), F32),
    )(p2, dinv, b2r, W3)

    p3 = _prop16_sc(src2d, dst2d, g3)

    g4 = pl.pallas_call(
        _b4_body,
        grid=(GRID,),
        in_specs=[_pairmat_spec(16), _mat_spec(16), _vec_spec(),
                  _full((1, 16)), _full((1, 16))],
        out_specs=_vec_spec(),
        out_shape=jax.ShapeDtypeStruct((NP,), F32),
    )(p3, g3, dinv, b3r, W4.reshape(1, 16))

    p4 = _prop1_sc(src2d, dst2d, g4)

    out = pl.pallas_call(
        _b5_body,
        grid=(GRID,),
        in_specs=[_pair_spec(), _vec_spec(), _vec_spec(), _full((1, 1))],
        out_specs=_vec_spec(),
        out_shape=jax.ShapeDtypeStruct((NP,), F32),
    )(p4, g4, dinv, b4r)

    return out[:N].reshape(N, 1)


# speedup vs baseline: 81.6570x; 2.1419x over previous
import functools

import jax
import jax.numpy as jnp
from jax import lax
from jax.experimental import pallas as pl
from jax.experimental.pallas import tpu as pltpu
from jax.experimental.pallas import tpu_sc as plsc

N = 50000
E = 1600000

NC = 2
NS = 16
NW = NC * NS

NP = 51200
STRIPE = NP // NS
NPS = 50176
STRIPE_S = NPS // NS
CH = 784

EPAD = 1638400
EPW = EPAD // NW
EPT = EPAD // NS
WR = 2560
F32 = jnp.float32

_MESH = plsc.VectorSubcoreMesh(core_axis_name="c", subcore_axis_name="s")
_PARAMS = pltpu.CompilerParams(use_tc_tiling_on_sc=False)


def _edge_pipeline(nwin, load_idx, fire_gather, drain_gather, fire_scatter,
                   drain_scatter):
    def step(t, b):
        drain_scatter(b)
        load_idx(t, b)
        fire_gather(b)
        drain_gather(1 - b)
        fire_scatter(1 - b)

    load_idx(0, 0)
    fire_gather(0)
    load_idx(1, 1)
    fire_gather(1)
    drain_gather(0)
    fire_scatter(0)

    @pl.loop(0, (nwin - 2) // 2)
    def _(i):
        step(2 * i + 2, 0)
        step(2 * i + 3, 1)

    drain_gather(1)
    fire_scatter(1)
    drain_scatter(0)
    drain_scatter(1)


@functools.partial(
    pl.kernel, mesh=_MESH, compiler_params=_PARAMS,
    out_type=jax.ShapeDtypeStruct((NC, NP), F32),
    scratch_types=[
        pltpu.VMEM((WR,), jnp.int32), pltpu.VMEM((WR,), jnp.int32),
        pltpu.VMEM((WR,), jnp.int32), pltpu.VMEM((WR,), jnp.int32),
        pltpu.VMEM((WR,), F32), pltpu.VMEM((WR,), F32),
        pltpu.VMEM((STRIPE,), F32),
        pltpu.VMEM_SHARED((NP,), F32),
        pltpu.VMEM_SHARED((NP,), F32),
        pltpu.SemaphoreType.DMA, pltpu.SemaphoreType.DMA,
        pltpu.SemaphoreType.DMA, pltpu.SemaphoreType.DMA,
    ])
def _prop1_sc(src_hbm, dst_hbm, g_hbm, out_hbm, sidx0, sidx1, didx0, didx1,
              rows0, rows1, stage_v, g_sp, acc_sp, gsem0, gsem1, ssem0,
              ssem1):
    c = lax.axis_index("c")
    s = lax.axis_index("s")
    st = pl.multiple_of(s * STRIPE, 128)
    pltpu.sync_copy(g_hbm.at[pl.ds(st, STRIPE)], stage_v)
    pltpu.sync_copy(stage_v, g_sp.at[pl.ds(st, STRIPE)])
    pltpu.sync_copy(stage_v, acc_sp.at[pl.ds(st, STRIPE)])
    plsc.subcore_barrier()

    e0 = (c * NS + s) * EPW
    sidx = (sidx0, sidx1)
    didx = (didx0, didx1)
    rows = (rows0, rows1)
    gsem = (gsem0, gsem1)
    ssem = (ssem0, ssem1)

    def load(t, b):
        off = pl.multiple_of(e0 + t * WR, 128)
        pltpu.sync_copy(src_hbm.at[pl.ds(off, WR)], sidx[b])
        pltpu.sync_copy(dst_hbm.at[pl.ds(off, WR)], didx[b])

    def fire_g(b):
        pltpu.async_copy(g_sp.at[sidx[b]], rows[b], gsem[b])

    def drain_g(b):
        pltpu.make_async_copy(g_sp.at[sidx[b]], rows[b], gsem[b]).wait()

    def fire_s(b):
        pltpu.async_copy(rows[b], acc_sp.at[didx[b]], ssem[b], add=True)

    def drain_s(b):
        pltpu.make_async_copy(rows[b], acc_sp.at[didx[b]], ssem[b]).wait()

    _edge_pipeline(EPW // WR, load, fire_g, drain_g, fire_s, drain_s)

    plsc.subcore_barrier()
    pltpu.sync_copy(acc_sp.at[pl.ds(st, STRIPE)], stage_v)
    pltpu.sync_copy(stage_v, out_hbm.at[c, pl.ds(st, STRIPE)])


def _make_propF(F, feature_split):
    @functools.partial(
        pl.kernel, mesh=_MESH, compiler_params=_PARAMS,
        out_type=jax.ShapeDtypeStruct((NC, NP, F), F32),
        scratch_types=[
            pltpu.VMEM((WR,), jnp.int32),
            pltpu.VMEM((WR,), jnp.int32),
            pltpu.VMEM((WR, F), F32),
            pltpu.VMEM((CH, F), F32),
            pltpu.VMEM_SHARED((NPS, F), F32),
        ])
    def k(src_hbm, dst_hbm, g_hbm, out_hbm, sidx, didx, rows_v, stage_v,
          acc_sp):
        c = lax.axis_index("c")
        s = lax.axis_index("s")
        sts = pl.multiple_of(s * STRIPE_S, 8)

        if feature_split:
            e0 = s * EPT
            nwin = EPT // WR
        else:
            e0 = (c * NS + s) * EPW
            nwin = EPW // WR

        g_src = g_hbm.at[c] if feature_split else g_hbm

        for j in range(STRIPE_S // CH):
            r = sts + j * CH
            pltpu.sync_copy(g_src.at[pl.ds(r, CH)], stage_v)
            pltpu.sync_copy(stage_v, acc_sp.at[pl.ds(r, CH)])
        plsc.subcore_barrier()

        @pl.loop(0, nwin)
        def _(t):
            off = pl.multiple_of(e0 + t * WR, 128)
            pltpu.sync_copy(src_hbm.at[pl.ds(off, WR)], sidx)
            pltpu.sync_copy(dst_hbm.at[pl.ds(off, WR)], didx)
            pltpu.sync_copy(g_src.at[sidx], rows_v)
            pltpu.sync_copy(rows_v, acc_sp.at[didx], add=True)

        plsc.subcore_barrier()
        for j in range(STRIPE_S // CH):
            r = sts + j * CH
            pltpu.sync_copy(acc_sp.at[pl.ds(r, CH)], stage_v)
            pltpu.sync_copy(stage_v, out_hbm.at[c, pl.ds(r, CH)])

    return k


_propf_sc = _make_propF(16, feature_split=False)
_prop32_sc = _make_propF(16, feature_split=True)


BN = 10240
GRID = NP // BN


def _b1_body(degp_ref, x_ref, dinv_ref, g1_ref):
    deg = degp_ref[0] + degp_ref[1] - 1.0
    dinv = lax.rsqrt(deg)
    dinv_ref[...] = dinv
    g1_ref[...] = dinv * x_ref[...]


def _b2_body(p1_ref, g1_ref, dinv_ref, w1_ref, b1_ref, w2_ref, out_ref):
    dinv = dinv_ref[...]
    p0 = dinv * (p1_ref[0] + p1_ref[1] - g1_ref[...])
    h1 = jnp.maximum(p0[:, None] * w1_ref[0][None, :] + b1_ref[...], 0.0)
    a2 = jnp.dot(h1, w2_ref[...], preferred_element_type=F32)
    g2 = dinv[:, None] * a2
    out_ref[0] = g2[:, :16]
    out_ref[1] = g2[:, 16:]


def _b3_body(p2_ref, dinv_ref, b2_ref, w3_ref, out_ref):
    dinv = dinv_ref[...]
    acc = jnp.concatenate([p2_ref[0], p2_ref[1]], axis=-1)
    h2 = jnp.maximum(dinv[:, None] * acc + b2_ref[...], 0.0)
    a3 = jnp.dot(h2, w3_ref[...], preferred_element_type=F32)
    out_ref[...] = dinv[:, None] * a3


def _b4_body(p3_ref, g3_ref, dinv_ref, b3_ref, w4_ref, out_ref):
    dinv = dinv_ref[...]
    acc = p3_ref[0] + p3_ref[1] - g3_ref[...]
    h3 = jnp.maximum(dinv[:, None] * acc + b3_ref[...], 0.0)
    a4 = jnp.sum(h3 * w4_ref[0][None, :], axis=-1)
    out_ref[...] = dinv * a4


def _b5_body(p4_ref, g4_ref, dinv_ref, b4_ref, out_ref):
    out_ref[...] = (dinv_ref[...] * (p4_ref[0] + p4_ref[1] - g4_ref[...])
                    + b4_ref[0, 0])


def _vec_spec():
    return pl.BlockSpec((BN,), lambda i: (i,))


def _pair_spec():
    return pl.BlockSpec((2, BN), lambda i: (0, i))


def _mat_spec(F):
    return pl.BlockSpec((BN, F), lambda i: (i, 0))


def _pairmat_spec(F):
    return pl.BlockSpec((2, BN, F), lambda i: (0, i, 0))


def _full(shape):
    return pl.BlockSpec(shape, lambda i: tuple(0 for _ in shape))


def kernel(x, edge_index, W1, b1, W2, b2, W3, b3, W4, b4):
    src = edge_index[0].astype(jnp.int32)
    dst = edge_index[1].astype(jnp.int32)

    pad_e = EPAD - E
    pad_ids = (N + (jnp.arange(pad_e, dtype=jnp.int32) % (NPS - N)))
    srcp = jnp.concatenate([src, pad_ids])
    dstp = jnp.concatenate([dst, pad_ids])

    xp = jnp.pad(x[:, 0], (0, NP - N))
    ones_n = jnp.ones((NP,), F32)
    b1r = b1.reshape(1, 64)
    b2r = b2.reshape(1, 32)
    b3r = b3.reshape(1, 16)
    b4r = b4.reshape(1, 1)

    degp = _prop1_sc(srcp, dstp, ones_n)

    dinv, g1 = pl.pallas_call(
        _b1_body,
        grid=(GRID,),
        in_specs=[_pair_spec(), _vec_spec()],
        out_specs=[_vec_spec(), _vec_spec()],
        out_shape=[jax.ShapeDtypeStruct((NP,), F32),
                   jax.ShapeDtypeStruct((NP,), F32)],
    )(degp, xp)

    p1 = _prop1_sc(srcp, dstp, g1)

    g2 = pl.pallas_call(
        _b2_body,
        grid=(GRID,),
        in_specs=[_pair_spec(), _vec_spec(), _vec_spec(),
                  _full((1, 64)), _full((1, 64)), _full((64, 32))],
        out_specs=_pairmat_spec(16),
        out_shape=jax.ShapeDtypeStruct((2, NP, 16), F32),
    )(p1, g1, dinv, W1, b1r, W2)

    p2 = _prop32_sc(srcp, dstp, g2)

    g3 = pl.pallas_call(
        _b3_body,
        grid=(GRID,),
        in_specs=[_pairmat_spec(16), _vec_spec(),
                  _full((1, 32)), _full((32, 16))],
        out_specs=_mat_spec(16),
        out_shape=jax.ShapeDtypeStruct((NP, 16), F32),
    )(p2, dinv, b2r, W3)

    p3 = _propf_sc(srcp, dstp, g3)

    g4 = pl.pallas_call(
        _b4_body,
        grid=(GRID,),
        in_specs=[_pairmat_spec(16), _mat_spec(16), _vec_spec(),
                  _full((1, 16)), _full((1, 16))],
        out_specs=_vec_spec(),
        out_shape=jax.ShapeDtypeStruct((NP,), F32),
    )(p3, g3, dinv, b3r, W4.reshape(1, 16))

    p4 = _prop1_sc(srcp, dstp, g4)

    out = pl.pallas_call(
        _b5_body,
        grid=(GRID,),
        in_specs=[_pair_spec(), _vec_spec(), _vec_spec(), _full((1, 1))],
        out_specs=_vec_spec(),
        out_shape=jax.ShapeDtypeStruct((NP,), F32),
    )(p4, g4, dinv, b4r)

    return out[:N].reshape(N, 1)


# speedup vs baseline: 83.9601x; 1.0282x over previous
import functools

import jax
import jax.numpy as jnp
from jax import lax
from jax.experimental import pallas as pl
from jax.experimental.pallas import tpu as pltpu
from jax.experimental.pallas import tpu_sc as plsc

N = 50000
E = 1600000

NC = 2
NS = 16
NW = NC * NS

NP = 51200
STRIPE = NP // NS
NPS = 50176
STRIPE_S = NPS // NS
CH = 784

EPAD = 1638400
EPW = EPAD // NW
EPT = EPAD // NS
WR = 3200
NSLAB = EPAD // WR
F32 = jnp.float32

_MESH = plsc.VectorSubcoreMesh(core_axis_name="c", subcore_axis_name="s")
_PARAMS = pltpu.CompilerParams(use_tc_tiling_on_sc=False)


def _edge_pipeline(nwin, load_idx, fire_gather, drain_gather, fire_scatter,
                   drain_scatter):
    def step(t, b):
        drain_scatter(b)
        load_idx(t, b)
        fire_gather(b)
        drain_gather(1 - b)
        fire_scatter(1 - b)

    load_idx(0, 0)
    fire_gather(0)
    load_idx(1, 1)
    fire_gather(1)
    drain_gather(0)
    fire_scatter(0)

    @pl.loop(0, (nwin - 2) // 2)
    def _(i):
        step(2 * i + 2, 0)
        step(2 * i + 3, 1)

    drain_gather(1)
    fire_scatter(1)
    drain_scatter(0)
    drain_scatter(1)


@functools.partial(
    pl.kernel, mesh=_MESH, compiler_params=_PARAMS,
    out_type=jax.ShapeDtypeStruct((NC, NP), F32),
    scratch_types=[
        pltpu.VMEM((2, WR), jnp.int32), pltpu.VMEM((2, WR), jnp.int32),
        pltpu.VMEM((WR,), F32), pltpu.VMEM((WR,), F32),
        pltpu.VMEM((STRIPE,), F32),
        pltpu.VMEM_SHARED((NP,), F32),
        pltpu.VMEM_SHARED((NP,), F32),
        pltpu.SemaphoreType.DMA, pltpu.SemaphoreType.DMA,
        pltpu.SemaphoreType.DMA, pltpu.SemaphoreType.DMA,
    ])
def _prop1_sc(eidx_hbm, g_hbm, out_hbm, slab0, slab1,
              rows0, rows1, stage_v, g_sp, acc_sp, gsem0, gsem1, ssem0,
              ssem1):
    c = lax.axis_index("c")
    s = lax.axis_index("s")
    st = pl.multiple_of(s * STRIPE, 128)
    pltpu.sync_copy(g_hbm.at[pl.ds(st, STRIPE)], stage_v)
    pltpu.sync_copy(stage_v, g_sp.at[pl.ds(st, STRIPE)])
    pltpu.sync_copy(stage_v, acc_sp.at[pl.ds(st, STRIPE)])
    plsc.subcore_barrier()

    w0 = (c * NS + s) * (EPW // WR)
    slab = (slab0, slab1)
    rows = (rows0, rows1)
    gsem = (gsem0, gsem1)
    ssem = (ssem0, ssem1)

    def load(t, b):
        pltpu.sync_copy(eidx_hbm.at[w0 + t], slab[b])

    def fire_g(b):
        pltpu.async_copy(g_sp.at[slab[b].at[0]], rows[b], gsem[b])

    def drain_g(b):
        pltpu.make_async_copy(g_sp.at[slab[b].at[0]], rows[b], gsem[b]).wait()

    def fire_s(b):
        pltpu.async_copy(rows[b], acc_sp.at[slab[b].at[1]], ssem[b], add=True)

    def drain_s(b):
        pltpu.make_async_copy(rows[b], acc_sp.at[slab[b].at[1]],
                              ssem[b]).wait()

    _edge_pipeline(EPW // WR, load, fire_g, drain_g, fire_s, drain_s)

    plsc.subcore_barrier()
    pltpu.sync_copy(acc_sp.at[pl.ds(st, STRIPE)], stage_v)
    pltpu.sync_copy(stage_v, out_hbm.at[c, pl.ds(st, STRIPE)])


def _make_propF(F, feature_split):
    @functools.partial(
        pl.kernel, mesh=_MESH, compiler_params=_PARAMS,
        out_type=jax.ShapeDtypeStruct((NC, NP, F), F32),
        scratch_types=[
            pltpu.VMEM((2, WR), jnp.int32),
            pltpu.VMEM((WR, F), F32),
            pltpu.VMEM((CH, F), F32),
            pltpu.VMEM_SHARED((NPS, F), F32),
        ])
    def k(eidx_hbm, g_hbm, out_hbm, slab_v, rows_v, stage_v, acc_sp):
        c = lax.axis_index("c")
        s = lax.axis_index("s")
        sts = pl.multiple_of(s * STRIPE_S, 8)

        if feature_split:
            w0 = s * (EPT // WR)
            nwin = EPT // WR
        else:
            w0 = (c * NS + s) * (EPW // WR)
            nwin = EPW // WR

        g_src = g_hbm.at[c] if feature_split else g_hbm

        for j in range(STRIPE_S // CH):
            r = sts + j * CH
            pltpu.sync_copy(g_src.at[pl.ds(r, CH)], stage_v)
            pltpu.sync_copy(stage_v, acc_sp.at[pl.ds(r, CH)])
        plsc.subcore_barrier()

        @pl.loop(0, nwin)
        def _(t):
            pltpu.sync_copy(eidx_hbm.at[w0 + t], slab_v)
            pltpu.sync_copy(g_src.at[slab_v.at[0]], rows_v)
            pltpu.sync_copy(rows_v, acc_sp.at[slab_v.at[1]], add=True)

        plsc.subcore_barrier()
        for j in range(STRIPE_S // CH):
            r = sts + j * CH
            pltpu.sync_copy(acc_sp.at[pl.ds(r, CH)], stage_v)
            pltpu.sync_copy(stage_v, out_hbm.at[c, pl.ds(r, CH)])

    return k


_propf_sc = _make_propF(16, feature_split=False)
_prop32_sc = _make_propF(16, feature_split=True)


BN = 10240
GRID = NP // BN


def _b1_body(degp_ref, x_ref, dinv_ref, g1_ref):
    deg = degp_ref[0] + degp_ref[1] - 1.0
    dinv = lax.rsqrt(deg)
    dinv_ref[...] = dinv
    g1_ref[...] = dinv * x_ref[...]


def _b2_body(p1_ref, g1_ref, dinv_ref, w1_ref, b1_ref, w2_ref, out_ref):
    dinv = dinv_ref[...]
    p0 = dinv * (p1_ref[0] + p1_ref[1] - g1_ref[...])
    h1 = jnp.maximum(p0[:, None] * w1_ref[0][None, :] + b1_ref[...], 0.0)
    a2 = jnp.dot(h1, w2_ref[...], preferred_element_type=F32)
    g2 = dinv[:, None] * a2
    out_ref[0] = g2[:, :16]
    out_ref[1] = g2[:, 16:]


def _b3_body(p2_ref, dinv_ref, b2_ref, w3_ref, out_ref):
    dinv = dinv_ref[...]
    acc = jnp.concatenate([p2_ref[0], p2_ref[1]], axis=-1)
    h2 = jnp.maximum(dinv[:, None] * acc + b2_ref[...], 0.0)
    a3 = jnp.dot(h2, w3_ref[...], preferred_element_type=F32)
    out_ref[...] = dinv[:, None] * a3


def _b4_body(p3_ref, g3_ref, dinv_ref, b3_ref, w4_ref, out_ref):
    dinv = dinv_ref[...]
    acc = p3_ref[0] + p3_ref[1] - g3_ref[...]
    h3 = jnp.maximum(dinv[:, None] * acc + b3_ref[...], 0.0)
    a4 = jnp.sum(h3 * w4_ref[0][None, :], axis=-1)
    out_ref[...] = dinv * a4


def _b5_body(p4_ref, g4_ref, dinv_ref, b4_ref, out_ref):
    out_ref[...] = (dinv_ref[...] * (p4_ref[0] + p4_ref[1] - g4_ref[...])
                    + b4_ref[0, 0])


def _vec_spec():
    return pl.BlockSpec((BN,), lambda i: (i,))


def _pair_spec():
    return pl.BlockSpec((2, BN), lambda i: (0, i))


def _mat_spec(F):
    return pl.BlockSpec((BN, F), lambda i: (i, 0))


def _pairmat_spec(F):
    return pl.BlockSpec((2, BN, F), lambda i: (0, i, 0))


def _full(shape):
    return pl.BlockSpec(shape, lambda i: tuple(0 for _ in shape))


def kernel(x, edge_index, W1, b1, W2, b2, W3, b3, W4, b4):
    src = edge_index[0].astype(jnp.int32)
    dst = edge_index[1].astype(jnp.int32)

    pad_e = EPAD - E
    pad_ids = (N + (jnp.arange(pad_e, dtype=jnp.int32) % (NPS - N)))
    srcp = jnp.concatenate([src, pad_ids])
    dstp = jnp.concatenate([dst, pad_ids])
    eidx = jnp.stack([srcp.reshape(NSLAB, WR), dstp.reshape(NSLAB, WR)],
                     axis=1)

    xp = jnp.pad(x[:, 0], (0, NP - N))
    ones_n = jnp.ones((NP,), F32)
    b1r = b1.reshape(1, 64)
    b2r = b2.reshape(1, 32)
    b3r = b3.reshape(1, 16)
    b4r = b4.reshape(1, 1)

    degp = _prop1_sc(eidx, ones_n)

    dinv, g1 = pl.pallas_call(
        _b1_body,
        grid=(GRID,),
        in_specs=[_pair_spec(), _vec_spec()],
        out_specs=[_vec_spec(), _vec_spec()],
        out_shape=[jax.ShapeDtypeStruct((NP,), F32),
                   jax.ShapeDtypeStruct((NP,), F32)],
    )(degp, xp)

    p1 = _prop1_sc(eidx, g1)

    g2 = pl.pallas_call(
        _b2_body,
        grid=(GRID,),
        in_specs=[_pair_spec(), _vec_spec(), _vec_spec(),
                  _full((1, 64)), _full((1, 64)), _full((64, 32))],
        out_specs=_pairmat_spec(16),
        out_shape=jax.ShapeDtypeStruct((2, NP, 16), F32),
    )(p1, g1, dinv, W1, b1r, W2)

    p2 = _prop32_sc(eidx, g2)

    g3 = pl.pallas_call(
        _b3_body,
        grid=(GRID,),
        in_specs=[_pairmat_spec(16), _vec_spec(),
                  _full((1, 32)), _full((32, 16))],
        out_specs=_mat_spec(16),
        out_shape=jax.ShapeDtypeStruct((NP, 16), F32),
    )(p2, dinv, b2r, W3)

    p3 = _propf_sc(eidx, g3)

    g4 = pl.pallas_call(
        _b4_body,
        grid=(GRID,),
        in_specs=[_pairmat_spec(16), _mat_spec(16), _vec_spec(),
                  _full((1, 16)), _full((1, 16))],
        out_specs=_vec_spec(),
        out_shape=jax.ShapeDtypeStruct((NP,), F32),
    )(p3, g3, dinv, b3r, W4.reshape(1, 16))

    p4 = _prop1_sc(eidx, g4)

    out = pl.pallas_call(
        _b5_body,
        grid=(GRID,),
        in_specs=[_pair_spec(), _vec_spec(), _vec_spec(), _full((1, 1))],
        out_specs=_vec_spec(),
        out_shape=jax.ShapeDtypeStruct((NP,), F32),
    )(p4, g4, dinv, b4r)

    return out[:N].reshape(N, 1)


# speedup vs baseline: 87.1227x; 1.0377x over previous
import functools

import jax
import jax.numpy as jnp
from jax import lax
from jax.experimental import pallas as pl
from jax.experimental.pallas import tpu as pltpu
from jax.experimental.pallas import tpu_sc as plsc

N = 50000
E = 1600000

NC = 2
NS = 16
NW = NC * NS

NP = 51200
STRIPE = NP // NS
NPS = 50176
STRIPE_S = NPS // NS
CH = 784

EPAD = 1638400
EPW = EPAD // NW
EPT = EPAD // NS
WR = 3200
NSLAB = EPAD // WR
F32 = jnp.float32

_MESH = plsc.VectorSubcoreMesh(core_axis_name="c", subcore_axis_name="s")
_PARAMS = pltpu.CompilerParams(use_tc_tiling_on_sc=False)


def _edge_pipeline(nwin, load_idx, fire_gather, drain_gather, fire_scatter,
                   drain_scatter):
    def step(t, b):
        drain_scatter(b)
        load_idx(t, b)
        fire_gather(b)
        drain_gather(1 - b)
        fire_scatter(1 - b)

    load_idx(0, 0)
    fire_gather(0)
    load_idx(1, 1)
    fire_gather(1)
    drain_gather(0)
    fire_scatter(0)

    @pl.loop(0, (nwin - 2) // 2)
    def _(i):
        step(2 * i + 2, 0)
        step(2 * i + 3, 1)

    drain_gather(1)
    fire_scatter(1)
    drain_scatter(0)
    drain_scatter(1)


@functools.partial(
    pl.kernel, mesh=_MESH, compiler_params=_PARAMS,
    out_type=jax.ShapeDtypeStruct((NC, NP), F32),
    scratch_types=[
        pltpu.VMEM((2, WR), jnp.int32), pltpu.VMEM((2, WR), jnp.int32),
        pltpu.VMEM((WR,), F32), pltpu.VMEM((WR,), F32),
        pltpu.VMEM((STRIPE,), F32),
        pltpu.VMEM_SHARED((NP,), F32),
        pltpu.VMEM_SHARED((NP,), F32),
        pltpu.SemaphoreType.DMA, pltpu.SemaphoreType.DMA,
        pltpu.SemaphoreType.DMA, pltpu.SemaphoreType.DMA,
    ])
def _prop1_sc(eidx_hbm, g_hbm, out_hbm, slab0, slab1,
              rows0, rows1, stage_v, g_sp, acc_sp, gsem0, gsem1, ssem0,
              ssem1):
    c = lax.axis_index("c")
    s = lax.axis_index("s")
    st = pl.multiple_of(s * STRIPE, 128)
    pltpu.sync_copy(g_hbm.at[pl.ds(st, STRIPE)], stage_v)
    pltpu.sync_copy(stage_v, g_sp.at[pl.ds(st, STRIPE)])
    pltpu.sync_copy(stage_v, acc_sp.at[pl.ds(st, STRIPE)])
    plsc.subcore_barrier()

    w0 = (c * NS + s) * (EPW // WR)
    slab = (slab0, slab1)
    rows = (rows0, rows1)
    gsem = (gsem0, gsem1)
    ssem = (ssem0, ssem1)

    def load(t, b):
        pltpu.sync_copy(eidx_hbm.at[w0 + t], slab[b])

    def fire_g(b):
        pltpu.async_copy(g_sp.at[slab[b].at[0]], rows[b], gsem[b])

    def drain_g(b):
        pltpu.make_async_copy(g_sp.at[slab[b].at[0]], rows[b], gsem[b]).wait()

    def fire_s(b):
        pltpu.async_copy(rows[b], acc_sp.at[slab[b].at[1]], ssem[b], add=True)

    def drain_s(b):
        pltpu.make_async_copy(rows[b], acc_sp.at[slab[b].at[1]],
                              ssem[b]).wait()

    _edge_pipeline(EPW // WR, load, fire_g, drain_g, fire_s, drain_s)

    plsc.subcore_barrier()
    pltpu.sync_copy(acc_sp.at[pl.ds(st, STRIPE)], stage_v)
    pltpu.sync_copy(stage_v, out_hbm.at[c, pl.ds(st, STRIPE)])


@functools.partial(
    pl.kernel, mesh=_MESH, compiler_params=_PARAMS,
    out_type=jax.ShapeDtypeStruct((NC, NP), F32),
    scratch_types=[
        pltpu.VMEM((2, WR), jnp.int32), pltpu.VMEM((2, WR), jnp.int32),
        pltpu.VMEM((WR,), F32),
        pltpu.VMEM((STRIPE,), F32),
        pltpu.VMEM_SHARED((NP,), F32),
        pltpu.SemaphoreType.DMA, pltpu.SemaphoreType.DMA,
    ])
def _deg_sc(eidx_hbm, zeros_hbm, out_hbm, slab0, slab1, ones_v, stage_v,
            acc_sp, ssem0, ssem1):
    c = lax.axis_index("c")
    s = lax.axis_index("s")
    st = pl.multiple_of(s * STRIPE, 128)
    pltpu.sync_copy(zeros_hbm.at[pl.ds(st, STRIPE)], stage_v)
    pltpu.sync_copy(stage_v, acc_sp.at[pl.ds(st, STRIPE)])
    one = jnp.full((16,), 1.0, F32)

    @pl.loop(0, WR // 16)
    def _(i):
        ones_v[pl.ds(i * 16, 16)] = one

    plsc.subcore_barrier()

    w0 = (c * NS + s) * (EPW // WR)
    slab = (slab0, slab1)
    ssem = (ssem0, ssem1)

    def load(t, b):
        pltpu.sync_copy(eidx_hbm.at[w0 + t], slab[b])

    def fire(b):
        pltpu.async_copy(ones_v, acc_sp.at[slab[b].at[1]], ssem[b], add=True)

    def drain(b):
        pltpu.make_async_copy(ones_v, acc_sp.at[slab[b].at[1]],
                              ssem[b]).wait()

    nwin = EPW // WR
    load(0, 0)
    fire(0)
    load(1, 1)
    fire(1)

    @pl.loop(0, (nwin - 2) // 2)
    def _(i):
        for b in (0, 1):
            drain(b)
            load(2 * i + 2 + b, b)
            fire(b)

    drain(0)
    drain(1)
    plsc.subcore_barrier()
    pltpu.sync_copy(acc_sp.at[pl.ds(st, STRIPE)], stage_v)
    pltpu.sync_copy(stage_v, out_hbm.at[c, pl.ds(st, STRIPE)])


def _make_propF(F, feature_split):
    @functools.partial(
        pl.kernel, mesh=_MESH, compiler_params=_PARAMS,
        out_type=jax.ShapeDtypeStruct((NC, NP, F), F32),
        scratch_types=[
            pltpu.VMEM((2, WR), jnp.int32),
            pltpu.VMEM((WR, F), F32),
            pltpu.VMEM((CH, F), F32),
            pltpu.VMEM_SHARED((NPS, F), F32),
        ])
    def k(eidx_hbm, g_hbm, out_hbm, slab_v, rows_v, stage_v, acc_sp):
        c = lax.axis_index("c")
        s = lax.axis_index("s")
        sts = pl.multiple_of(s * STRIPE_S, 8)

        if feature_split:
            w0 = s * (EPT // WR)
            nwin = EPT // WR
        else:
            w0 = (c * NS + s) * (EPW // WR)
            nwin = EPW // WR

        g_src = g_hbm.at[c] if feature_split else g_hbm

        for j in range(STRIPE_S // CH):
            r = sts + j * CH
            pltpu.sync_copy(g_src.at[pl.ds(r, CH)], stage_v)
            pltpu.sync_copy(stage_v, acc_sp.at[pl.ds(r, CH)])
        plsc.subcore_barrier()

        @pl.loop(0, nwin)
        def _(t):
            pltpu.sync_copy(eidx_hbm.at[w0 + t], slab_v)
            pltpu.sync_copy(g_src.at[slab_v.at[0]], rows_v)
            pltpu.sync_copy(rows_v, acc_sp.at[slab_v.at[1]], add=True)

        plsc.subcore_barrier()
        for j in range(STRIPE_S // CH):
            r = sts + j * CH
            pltpu.sync_copy(acc_sp.at[pl.ds(r, CH)], stage_v)
            pltpu.sync_copy(stage_v, out_hbm.at[c, pl.ds(r, CH)])

    return k


_propf_sc = _make_propF(16, feature_split=False)
_prop32_sc = _make_propF(16, feature_split=True)


BN = 10240
GRID = NP // BN


def _b1_body(degp_ref, x_ref, dinv_ref, g1_ref):
    deg = degp_ref[0] + degp_ref[1] + 1.0
    dinv = lax.rsqrt(deg)
    dinv_ref[...] = dinv
    g1_ref[...] = dinv * x_ref[...]


def _b2_body(p1_ref, g1_ref, dinv_ref, w1_ref, b1_ref, w2_ref, out_ref):
    dinv = dinv_ref[...]
    p0 = dinv * (p1_ref[0] + p1_ref[1] - g1_ref[...])
    h1 = jnp.maximum(p0[:, None] * w1_ref[0][None, :] + b1_ref[...], 0.0)
    a2 = jnp.dot(h1, w2_ref[...], preferred_element_type=F32)
    g2 = dinv[:, None] * a2
    out_ref[0] = g2[:, :16]
    out_ref[1] = g2[:, 16:]


def _b3_body(p2_ref, dinv_ref, b2_ref, w3_ref, out_ref):
    dinv = dinv_ref[...]
    acc = jnp.concatenate([p2_ref[0], p2_ref[1]], axis=-1)
    h2 = jnp.maximum(dinv[:, None] * acc + b2_ref[...], 0.0)
    a3 = jnp.dot(h2, w3_ref[...], preferred_element_type=F32)
    out_ref[...] = dinv[:, None] * a3


def _b4_body(p3_ref, g3_ref, dinv_ref, b3_ref, w4_ref, out_ref):
    dinv = dinv_ref[...]
    acc = p3_ref[0] + p3_ref[1] - g3_ref[...]
    h3 = jnp.maximum(dinv[:, None] * acc + b3_ref[...], 0.0)
    a4 = jnp.sum(h3 * w4_ref[0][None, :], axis=-1)
    out_ref[...] = dinv * a4


def _b5_body(p4_ref, g4_ref, dinv_ref, b4_ref, out_ref):
    out_ref[...] = (dinv_ref[...] * (p4_ref[0] + p4_ref[1] - g4_ref[...])
                    + b4_ref[0, 0])


def _vec_spec():
    return pl.BlockSpec((BN,), lambda i: (i,))


def _pair_spec():
    return pl.BlockSpec((2, BN), lambda i: (0, i))


def _mat_spec(F):
    return pl.BlockSpec((BN, F), lambda i: (i, 0))


def _pairmat_spec(F):
    return pl.BlockSpec((2, BN, F), lambda i: (0, i, 0))


def _full(shape):
    return pl.BlockSpec(shape, lambda i: tuple(0 for _ in shape))


def kernel(x, edge_index, W1, b1, W2, b2, W3, b3, W4, b4):
    src = edge_index[0].astype(jnp.int32)
    dst = edge_index[1].astype(jnp.int32)

    pad_e = EPAD - E
    pad_ids = (N + (jnp.arange(pad_e, dtype=jnp.int32) % (NPS - N)))
    srcp = jnp.concatenate([src, pad_ids])
    dstp = jnp.concatenate([dst, pad_ids])
    eidx = jnp.stack([srcp.reshape(NSLAB, WR), dstp.reshape(NSLAB, WR)],
                     axis=1)

    xp = jnp.pad(x[:, 0], (0, NP - N))
    zeros_n = jnp.zeros((NP,), F32)
    b1r = b1.reshape(1, 64)
    b2r = b2.reshape(1, 32)
    b3r = b3.reshape(1, 16)
    b4r = b4.reshape(1, 1)

    degp = _deg_sc(eidx, zeros_n)

    dinv, g1 = pl.pallas_call(
        _b1_body,
        grid=(GRID,),
        in_specs=[_pair_spec(), _vec_spec()],
        out_specs=[_vec_spec(), _vec_spec()],
        out_shape=[jax.ShapeDtypeStruct((NP,), F32),
                   jax.ShapeDtypeStruct((NP,), F32)],
    )(degp, xp)

    p1 = _prop1_sc(eidx, g1)

    g2 = pl.pallas_call(
        _b2_body,
        grid=(GRID,),
        in_specs=[_pair_spec(), _vec_spec(), _vec_spec(),
                  _full((1, 64)), _full((1, 64)), _full((64, 32))],
        out_specs=_pairmat_spec(16),
        out_shape=jax.ShapeDtypeStruct((2, NP, 16), F32),
    )(p1, g1, dinv, W1, b1r, W2)

    p2 = _prop32_sc(eidx, g2)

    g3 = pl.pallas_call(
        _b3_body,
        grid=(GRID,),
        in_specs=[_pairmat_spec(16), _vec_spec(),
                  _full((1, 32)), _full((32, 16))],
        out_specs=_mat_spec(16),
        out_shape=jax.ShapeDtypeStruct((NP, 16), F32),
    )(p2, dinv, b2r, W3)

    p3 = _propf_sc(eidx, g3)

    g4 = pl.pallas_call(
        _b4_body,
        grid=(GRID,),
        in_specs=[_pairmat_spec(16), _mat_spec(16), _vec_spec(),
                  _full((1, 16)), _full((1, 16))],
        out_specs=_vec_spec(),
        out_shape=jax.ShapeDtypeStruct((NP,), F32),
    )(p3, g3, dinv, b3r, W4.reshape(1, 16))

    p4 = _prop1_sc(eidx, g4)

    out = pl.pallas_call(
        _b5_body,
        grid=(GRID,),
        in_specs=[_pair_spec(), _vec_spec(), _vec_spec(), _full((1, 1))],
        out_specs=_vec_spec(),
        out_shape=jax.ShapeDtypeStruct((NP,), F32),
    )(p4, g4, dinv, b4r)

    return out[:N].reshape(N, 1)


# speedup vs baseline: 87.2317x; 1.0013x over previous
import functools

import jax
import jax.numpy as jnp
from jax import lax
from jax.experimental import pallas as pl
from jax.experimental.pallas import tpu as pltpu
from jax.experimental.pallas import tpu_sc as plsc

N = 50000
E = 1600000

NC = 2
NS = 16
NW = NC * NS

NP = 51200
STRIPE = NP // NS
NPS = 50176
STRIPE_S = NPS // NS
CH = 784

EPAD = 1638400
EPW = EPAD // NW
EPT = EPAD // NS
WR = 3200
NSLAB = EPAD // WR
F32 = jnp.float32

_MESH = plsc.VectorSubcoreMesh(core_axis_name="c", subcore_axis_name="s")
_PARAMS = pltpu.CompilerParams(use_tc_tiling_on_sc=False)


def _edge_pipeline(nwin, load_idx, fire_gather, drain_gather, fire_scatter,
                   drain_scatter):
    def step(t, b):
        drain_scatter(b)
        load_idx(t, b)
        fire_gather(b)
        drain_gather(1 - b)
        fire_scatter(1 - b)

    load_idx(0, 0)
    fire_gather(0)
    load_idx(1, 1)
    fire_gather(1)
    drain_gather(0)
    fire_scatter(0)

    @pl.loop(0, (nwin - 2) // 2)
    def _(i):
        step(2 * i + 2, 0)
        step(2 * i + 3, 1)

    drain_gather(1)
    fire_scatter(1)
    drain_scatter(0)
    drain_scatter(1)


@functools.partial(
    pl.kernel, mesh=_MESH, compiler_params=_PARAMS,
    out_type=jax.ShapeDtypeStruct((NC, NP), F32),
    scratch_types=[
        pltpu.VMEM((2, WR), jnp.int32), pltpu.VMEM((2, WR), jnp.int32),
        pltpu.VMEM((WR,), F32), pltpu.VMEM((WR,), F32),
        pltpu.VMEM((STRIPE,), F32),
        pltpu.VMEM_SHARED((NP,), F32),
        pltpu.VMEM_SHARED((NP,), F32),
        pltpu.SemaphoreType.DMA, pltpu.SemaphoreType.DMA,
        pltpu.SemaphoreType.DMA, pltpu.SemaphoreType.DMA,
    ])
def _prop1_sc(eidx_hbm, g_hbm, out_hbm, slab0, slab1,
              rows0, rows1, stage_v, g_sp, acc_sp, gsem0, gsem1, ssem0,
              ssem1):
    c = lax.axis_index("c")
    s = lax.axis_index("s")
    st = pl.multiple_of(s * STRIPE, 128)
    pltpu.sync_copy(g_hbm.at[pl.ds(st, STRIPE)], stage_v)
    pltpu.sync_copy(stage_v, g_sp.at[pl.ds(st, STRIPE)])
    pltpu.sync_copy(stage_v, acc_sp.at[pl.ds(st, STRIPE)])
    plsc.subcore_barrier()

    w0 = (c * NS + s) * (EPW // WR)
    slab = (slab0, slab1)
    rows = (rows0, rows1)
    gsem = (gsem0, gsem1)
    ssem = (ssem0, ssem1)

    def load(t, b):
        pltpu.sync_copy(eidx_hbm.at[w0 + t], slab[b])

    def fire_g(b):
        pltpu.async_copy(g_sp.at[slab[b].at[0]], rows[b], gsem[b])

    def drain_g(b):
        pltpu.make_async_copy(g_sp.at[slab[b].at[0]], rows[b], gsem[b]).wait()

    def fire_s(b):
        pltpu.async_copy(rows[b], acc_sp.at[slab[b].at[1]], ssem[b], add=True)

    def drain_s(b):
        pltpu.make_async_copy(rows[b], acc_sp.at[slab[b].at[1]],
                              ssem[b]).wait()

    _edge_pipeline(EPW // WR, load, fire_g, drain_g, fire_s, drain_s)

    plsc.subcore_barrier()
    pltpu.sync_copy(acc_sp.at[pl.ds(st, STRIPE)], stage_v)
    pltpu.sync_copy(stage_v, out_hbm.at[c, pl.ds(st, STRIPE)])


@functools.partial(
    pl.kernel, mesh=_MESH, compiler_params=_PARAMS,
    out_type=[jax.ShapeDtypeStruct((NC, NP), F32),
              jax.ShapeDtypeStruct((NP,), F32),
              jax.ShapeDtypeStruct((NP,), F32)],
    scratch_types=[
        pltpu.VMEM((2, WR), jnp.int32), pltpu.VMEM((2, WR), jnp.int32),
        pltpu.VMEM((WR,), F32), pltpu.VMEM((WR,), F32),
        pltpu.VMEM((STRIPE,), F32), pltpu.VMEM((STRIPE,), F32),
        pltpu.VMEM((STRIPE,), F32), pltpu.VMEM((STRIPE,), F32),
        pltpu.VMEM_SHARED((NP,), F32),
        pltpu.VMEM_SHARED((NP,), F32),
        pltpu.SemaphoreType.DMA, pltpu.SemaphoreType.DMA,
        pltpu.SemaphoreType.DMA, pltpu.SemaphoreType.DMA,
    ])
def _prop1x_sc(eidx_hbm, degp_hbm, x_hbm, outp_hbm, dinv_hbm, g1_hbm,
               slab0, slab1, rows0, rows1, d0_v, d1_v, dinv_v, g1_v,
               g_sp, acc_sp, gsem0, gsem1, ssem0, ssem1):
    c = lax.axis_index("c")
    s = lax.axis_index("s")
    st = pl.multiple_of(s * STRIPE, 128)

    pltpu.sync_copy(degp_hbm.at[0, pl.ds(st, STRIPE)], d0_v)
    pltpu.sync_copy(degp_hbm.at[1, pl.ds(st, STRIPE)], d1_v)
    pltpu.sync_copy(x_hbm.at[pl.ds(st, STRIPE)], g1_v)

    half = jnp.full((16,), 0.5, F32)
    three_half = jnp.full((16,), 1.5, F32)
    magic = jnp.full((16,), 0x5F3759DF, jnp.int32)
    sone = jnp.full((16,), 1, jnp.int32)

    @pl.loop(0, STRIPE // 16)
    def _(j):
        sl = pl.ds(j * 16, 16)
        d = d0_v[sl] + d1_v[sl] + 1.0
        y = lax.bitcast_convert_type(
            magic - lax.shift_right_arithmetic(
                lax.bitcast_convert_type(d, jnp.int32), sone), F32)
        y = y * (three_half - half * d * y * y)
        y = y * (three_half - half * d * y * y)
        y = y * (three_half - half * d * y * y)
        dinv_v[sl] = y
        g1_v[sl] = y * g1_v[sl]

    pltpu.sync_copy(g1_v, g_sp.at[pl.ds(st, STRIPE)])
    pltpu.sync_copy(g1_v, acc_sp.at[pl.ds(st, STRIPE)])

    @pl.when(c == 0)
    def _():
        pltpu.sync_copy(dinv_v, dinv_hbm.at[pl.ds(st, STRIPE)])
        pltpu.sync_copy(g1_v, g1_hbm.at[pl.ds(st, STRIPE)])

    plsc.subcore_barrier()

    w0 = (c * NS + s) * (EPW // WR)
    slab = (slab0, slab1)
    rows = (rows0, rows1)
    gsem = (gsem0, gsem1)
    ssem = (ssem0, ssem1)

    def load(t, b):
        pltpu.sync_copy(eidx_hbm.at[w0 + t], slab[b])

    def fire_g(b):
        pltpu.async_copy(g_sp.at[slab[b].at[0]], rows[b], gsem[b])

    def drain_g(b):
        pltpu.make_async_copy(g_sp.at[slab[b].at[0]], rows[b], gsem[b]).wait()

    def fire_s(b):
        pltpu.async_copy(rows[b], acc_sp.at[slab[b].at[1]], ssem[b], add=True)

    def drain_s(b):
        pltpu.make_async_copy(rows[b], acc_sp.at[slab[b].at[1]],
                              ssem[b]).wait()

    _edge_pipeline(EPW // WR, load, fire_g, drain_g, fire_s, drain_s)

    plsc.subcore_barrier()
    pltpu.sync_copy(acc_sp.at[pl.ds(st, STRIPE)], d0_v)
    pltpu.sync_copy(d0_v, outp_hbm.at[c, pl.ds(st, STRIPE)])


@functools.partial(
    pl.kernel, mesh=_MESH, compiler_params=_PARAMS,
    out_type=jax.ShapeDtypeStruct((NC, NP), F32),
    scratch_types=[
        pltpu.VMEM((2, WR), jnp.int32), pltpu.VMEM((2, WR), jnp.int32),
        pltpu.VMEM((WR,), F32),
        pltpu.VMEM((STRIPE,), F32),
        pltpu.VMEM_SHARED((NP,), F32),
        pltpu.SemaphoreType.DMA, pltpu.SemaphoreType.DMA,
    ])
def _deg_sc(eidx_hbm, zeros_hbm, out_hbm, slab0, slab1, ones_v, stage_v,
            acc_sp, ssem0, ssem1):
    c = lax.axis_index("c")
    s = lax.axis_index("s")
    st = pl.multiple_of(s * STRIPE, 128)
    pltpu.sync_copy(zeros_hbm.at[pl.ds(st, STRIPE)], stage_v)
    pltpu.sync_copy(stage_v, acc_sp.at[pl.ds(st, STRIPE)])
    one = jnp.full((16,), 1.0, F32)

    @pl.loop(0, WR // 16)
    def _(i):
        ones_v[pl.ds(i * 16, 16)] = one

    plsc.subcore_barrier()

    w0 = (c * NS + s) * (EPW // WR)
    slab = (slab0, slab1)
    ssem = (ssem0, ssem1)

    def load(t, b):
        pltpu.sync_copy(eidx_hbm.at[w0 + t], slab[b])

    def fire(b):
        pltpu.async_copy(ones_v, acc_sp.at[slab[b].at[1]], ssem[b], add=True)

    def drain(b):
        pltpu.make_async_copy(ones_v, acc_sp.at[slab[b].at[1]],
                              ssem[b]).wait()

    nwin = EPW // WR
    load(0, 0)
    fire(0)
    load(1, 1)
    fire(1)

    @pl.loop(0, (nwin - 2) // 2)
    def _(i):
        for b in (0, 1):
            drain(b)
            load(2 * i + 2 + b, b)
            fire(b)

    drain(0)
    drain(1)
    plsc.subcore_barrier()
    pltpu.sync_copy(acc_sp.at[pl.ds(st, STRIPE)], stage_v)
    pltpu.sync_copy(stage_v, out_hbm.at[c, pl.ds(st, STRIPE)])


def _make_propF(F, feature_split):
    @functools.partial(
        pl.kernel, mesh=_MESH, compiler_params=_PARAMS,
        out_type=jax.ShapeDtypeStruct((NC, NP, F), F32),
        scratch_types=[
            pltpu.VMEM((2, WR), jnp.int32),
            pltpu.VMEM((WR, F), F32),
            pltpu.VMEM((CH, F), F32),
            pltpu.VMEM_SHARED((NPS, F), F32),
        ])
    def k(eidx_hbm, g_hbm, out_hbm, slab_v, rows_v, stage_v, acc_sp):
        c = lax.axis_index("c")
        s = lax.axis_index("s")
        sts = pl.multiple_of(s * STRIPE_S, 8)

        if feature_split:
            w0 = s * (EPT // WR)
            nwin = EPT // WR
        else:
            w0 = (c * NS + s) * (EPW // WR)
            nwin = EPW // WR

        g_src = g_hbm.at[c] if feature_split else g_hbm

        for j in range(STRIPE_S // CH):
            r = sts + j * CH
            pltpu.sync_copy(g_src.at[pl.ds(r, CH)], stage_v)
            pltpu.sync_copy(stage_v, acc_sp.at[pl.ds(r, CH)])
        plsc.subcore_barrier()

        @pl.loop(0, nwin)
        def _(t):
            pltpu.sync_copy(eidx_hbm.at[w0 + t], slab_v)
            pltpu.sync_copy(g_src.at[slab_v.at[0]], rows_v)
            pltpu.sync_copy(rows_v, acc_sp.at[slab_v.at[1]], add=True)

        plsc.subcore_barrier()
        for j in range(STRIPE_S // CH):
            r = sts + j * CH
            pltpu.sync_copy(acc_sp.at[pl.ds(r, CH)], stage_v)
            pltpu.sync_copy(stage_v, out_hbm.at[c, pl.ds(r, CH)])

    return k


_propf_sc = _make_propF(16, feature_split=False)
_prop32_sc = _make_propF(16, feature_split=True)


BN = 10240
GRID = NP // BN


def _b1_body(degp_ref, x_ref, dinv_ref, g1_ref):
    deg = degp_ref[0] + degp_ref[1] + 1.0
    dinv = lax.rsqrt(deg)
    dinv_ref[...] = dinv
    g1_ref[...] = dinv * x_ref[...]


def _b2_body(p1_ref, g1_ref, dinv_ref, w1_ref, b1_ref, w2_ref, out_ref):
    dinv = dinv_ref[...]
    p0 = dinv * (p1_ref[0] + p1_ref[1] - g1_ref[...])
    h1 = jnp.maximum(p0[:, None] * w1_ref[0][None, :] + b1_ref[...], 0.0)
    a2 = jnp.dot(h1, w2_ref[...], preferred_element_type=F32)
    g2 = dinv[:, None] * a2
    out_ref[0] = g2[:, :16]
    out_ref[1] = g2[:, 16:]


def _b3_body(p2_ref, dinv_ref, b2_ref, w3_ref, out_ref):
    dinv = dinv_ref[...]
    acc = jnp.concatenate([p2_ref[0], p2_ref[1]], axis=-1)
    h2 = jnp.maximum(dinv[:, None] * acc + b2_ref[...], 0.0)
    a3 = jnp.dot(h2, w3_ref[...], preferred_element_type=F32)
    out_ref[...] = dinv[:, None] * a3


def _b4_body(p3_ref, g3_ref, dinv_ref, b3_ref, w4_ref, out_ref):
    dinv = dinv_ref[...]
    acc = p3_ref[0] + p3_ref[1] - g3_ref[...]
    h3 = jnp.maximum(dinv[:, None] * acc + b3_ref[...], 0.0)
    a4 = jnp.sum(h3 * w4_ref[0][None, :], axis=-1)
    out_ref[...] = dinv * a4


def _b5_body(p4_ref, g4_ref, dinv_ref, b4_ref, out_ref):
    out_ref[...] = (dinv_ref[...] * (p4_ref[0] + p4_ref[1] - g4_ref[...])
                    + b4_ref[0, 0])


def _vec_spec():
    return pl.BlockSpec((BN,), lambda i: (i,))


def _pair_spec():
    return pl.BlockSpec((2, BN), lambda i: (0, i))


def _mat_spec(F):
    return pl.BlockSpec((BN, F), lambda i: (i, 0))


def _pairmat_spec(F):
    return pl.BlockSpec((2, BN, F), lambda i: (0, i, 0))


def _full(shape):
    return pl.BlockSpec(shape, lambda i: tuple(0 for _ in shape))


def kernel(x, edge_index, W1, b1, W2, b2, W3, b3, W4, b4):
    src = edge_index[0].astype(jnp.int32)
    dst = edge_index[1].astype(jnp.int32)

    pad_e = EPAD - E
    pad_ids = (N + (jnp.arange(pad_e, dtype=jnp.int32) % (NPS - N)))
    srcp = jnp.concatenate([src, pad_ids])
    dstp = jnp.concatenate([dst, pad_ids])
    eidx = jnp.stack([srcp.reshape(NSLAB, WR), dstp.reshape(NSLAB, WR)],
                     axis=1)

    xp = jnp.pad(x[:, 0], (0, NP - N))
    zeros_n = jnp.zeros((NP,), F32)
    b1r = b1.reshape(1, 64)
    b2r = b2.reshape(1, 32)
    b3r = b3.reshape(1, 16)
    b4r = b4.reshape(1, 1)

    degp = _deg_sc(eidx, zeros_n)
    p1, dinv, g1 = _prop1x_sc(eidx, degp, xp)

    g2 = pl.pallas_call(
        _b2_body,
        grid=(GRID,),
        in_specs=[_pair_spec(), _vec_spec(), _vec_spec(),
                  _full((1, 64)), _full((1, 64)), _full((64, 32))],
        out_specs=_pairmat_spec(16),
        out_shape=jax.ShapeDtypeStruct((2, NP, 16), F32),
    )(p1, g1, dinv, W1, b1r, W2)

    p2 = _prop32_sc(eidx, g2)

    g3 = pl.pallas_call(
        _b3_body,
        grid=(GRID,),
        in_specs=[_pairmat_spec(16), _vec_spec(),
                  _full((1, 32)), _full((32, 16))],
        out_specs=_mat_spec(16),
        out_shape=jax.ShapeDtypeStruct((NP, 16), F32),
    )(p2, dinv, b2r, W3)

    p3 = _propf_sc(eidx, g3)

    g4 = pl.pallas_call(
        _b4_body,
        grid=(GRID,),
        in_specs=[_pairmat_spec(16), _mat_spec(16), _vec_spec(),
                  _full((1, 16)), _full((1, 16))],
        out_specs=_vec_spec(),
        out_shape=jax.ShapeDtypeStruct((NP,), F32),
    )(p3, g3, dinv, b3r, W4.reshape(1, 16))

    p4 = _prop1_sc(eidx, g4)

    out = pl.pallas_call(
        _b5_body,
        grid=(GRID,),
        in_specs=[_pair_spec(), _vec_spec(), _vec_spec(), _full((1, 1))],
        out_specs=_vec_spec(),
        out_shape=jax.ShapeDtypeStruct((NP,), F32),
    )(p4, g4, dinv, b4r)

    return out[:N].reshape(N, 1)


# speedup vs baseline: 87.2612x; 1.0003x over previous
import functools

import jax
import jax.numpy as jnp
from jax import lax
from jax.experimental import pallas as pl
from jax.experimental.pallas import tpu as pltpu
from jax.experimental.pallas import tpu_sc as plsc

N = 50000
E = 1600000

NC = 2
NS = 16
NW = NC * NS

NP = 51200
STRIPE = NP // NS
NPS = 50176
STRIPE_S = NPS // NS
CH = 784

EPAD = 1638400
EPW = EPAD // NW
EPT = EPAD // NS
WR = 3200
NSLAB = EPAD // WR
F32 = jnp.float32

_MESH = plsc.VectorSubcoreMesh(core_axis_name="c", subcore_axis_name="s")
_PARAMS = pltpu.CompilerParams(use_tc_tiling_on_sc=False)


def _edge_pipeline(nwin, load_idx, fire_gather, drain_gather, fire_scatter,
                   drain_scatter):
    def step(t, b):
        drain_scatter(b)
        load_idx(t, b)
        fire_gather(b)
        drain_gather(1 - b)
        fire_scatter(1 - b)

    load_idx(0, 0)
    fire_gather(0)
    load_idx(1, 1)
    fire_gather(1)
    drain_gather(0)
    fire_scatter(0)

    @pl.loop(0, (nwin - 2) // 2)
    def _(i):
        step(2 * i + 2, 0)
        step(2 * i + 3, 1)

    drain_gather(1)
    fire_scatter(1)
    drain_scatter(0)
    drain_scatter(1)


@functools.partial(
    pl.kernel, mesh=_MESH, compiler_params=_PARAMS,
    out_type=jax.ShapeDtypeStruct((NC, NP), F32),
    scratch_types=[
        pltpu.VMEM((2, WR), jnp.int32), pltpu.VMEM((2, WR), jnp.int32),
        pltpu.VMEM((WR,), F32), pltpu.VMEM((WR,), F32),
        pltpu.VMEM((STRIPE,), F32),
        pltpu.VMEM_SHARED((NP,), F32),
        pltpu.VMEM_SHARED((NP,), F32),
        pltpu.SemaphoreType.DMA, pltpu.SemaphoreType.DMA,
        pltpu.SemaphoreType.DMA, pltpu.SemaphoreType.DMA,
    ])
def _prop1_sc(eidx_hbm, g_hbm, out_hbm, slab0, slab1,
              rows0, rows1, stage_v, g_sp, acc_sp, gsem0, gsem1, ssem0,
              ssem1):
    c = lax.axis_index("c")
    s = lax.axis_index("s")
    st = pl.multiple_of(s * STRIPE, 128)
    pltpu.sync_copy(g_hbm.at[pl.ds(st, STRIPE)], stage_v)
    pltpu.sync_copy(stage_v, g_sp.at[pl.ds(st, STRIPE)])
    pltpu.sync_copy(stage_v, acc_sp.at[pl.ds(st, STRIPE)])
    plsc.subcore_barrier()

    w0 = (c * NS + s) * (EPW // WR)
    slab = (slab0, slab1)
    rows = (rows0, rows1)
    gsem = (gsem0, gsem1)
    ssem = (ssem0, ssem1)

    def load(t, b):
        pltpu.sync_copy(eidx_hbm.at[w0 + t], slab[b])

    def fire_g(b):
        pltpu.async_copy(g_sp.at[slab[b].at[0]], rows[b], gsem[b])

    def drain_g(b):
        pltpu.make_async_copy(g_sp.at[slab[b].at[0]], rows[b], gsem[b]).wait()

    def fire_s(b):
        pltpu.async_copy(rows[b], acc_sp.at[slab[b].at[1]], ssem[b], add=True)

    def drain_s(b):
        pltpu.make_async_copy(rows[b], acc_sp.at[slab[b].at[1]],
                              ssem[b]).wait()

    _edge_pipeline(EPW // WR, load, fire_g, drain_g, fire_s, drain_s)

    plsc.subcore_barrier()
    pltpu.sync_copy(acc_sp.at[pl.ds(st, STRIPE)], stage_v)
    pltpu.sync_copy(stage_v, out_hbm.at[c, pl.ds(st, STRIPE)])


@functools.partial(
    pl.kernel, mesh=_MESH, compiler_params=_PARAMS,
    out_type=[jax.ShapeDtypeStruct((NC, NP), F32),
              jax.ShapeDtypeStruct((NP,), F32),
              jax.ShapeDtypeStruct((NP,), F32)],
    scratch_types=[
        pltpu.VMEM((2, WR), jnp.int32), pltpu.VMEM((2, WR), jnp.int32),
        pltpu.VMEM((WR,), F32), pltpu.VMEM((WR,), F32),
        pltpu.VMEM((STRIPE,), F32), pltpu.VMEM((STRIPE,), F32),
        pltpu.VMEM((STRIPE,), F32), pltpu.VMEM((STRIPE,), F32),
        pltpu.VMEM_SHARED((NP,), F32),
        pltpu.VMEM_SHARED((NP,), F32),
        pltpu.SemaphoreType.DMA, pltpu.SemaphoreType.DMA,
        pltpu.SemaphoreType.DMA, pltpu.SemaphoreType.DMA,
    ])
def _prop1x_sc(eidx_hbm, degp_hbm, x_hbm, outp_hbm, dinv_hbm, g1_hbm,
               slab0, slab1, rows0, rows1, d0_v, d1_v, dinv_v, g1_v,
               g_sp, acc_sp, gsem0, gsem1, ssem0, ssem1):
    c = lax.axis_index("c")
    s = lax.axis_index("s")
    st = pl.multiple_of(s * STRIPE, 128)

    pltpu.sync_copy(degp_hbm.at[0, pl.ds(st, STRIPE)], d0_v)
    pltpu.sync_copy(degp_hbm.at[1, pl.ds(st, STRIPE)], d1_v)
    pltpu.sync_copy(x_hbm.at[pl.ds(st, STRIPE)], g1_v)

    half = jnp.full((16,), 0.5, F32)
    three_half = jnp.full((16,), 1.5, F32)
    magic = jnp.full((16,), 0x5F3759DF, jnp.int32)
    sone = jnp.full((16,), 1, jnp.int32)

    @pl.loop(0, STRIPE // 16)
    def _(j):
        sl = pl.ds(j * 16, 16)
        d = d0_v[sl] + d1_v[sl] + 1.0
        y = lax.bitcast_convert_type(
            magic - lax.shift_right_arithmetic(
                lax.bitcast_convert_type(d, jnp.int32), sone), F32)
        y = y * (three_half - half * d * y * y)
        y = y * (three_half - half * d * y * y)
        y = y * (three_half - half * d * y * y)
        dinv_v[sl] = y
        g1_v[sl] = y * g1_v[sl]

    pltpu.sync_copy(g1_v, g_sp.at[pl.ds(st, STRIPE)])
    pltpu.sync_copy(g1_v, acc_sp.at[pl.ds(st, STRIPE)])

    @pl.when(c == 0)
    def _():
        pltpu.sync_copy(dinv_v, dinv_hbm.at[pl.ds(st, STRIPE)])
        pltpu.sync_copy(g1_v, g1_hbm.at[pl.ds(st, STRIPE)])

    plsc.subcore_barrier()

    w0 = (c * NS + s) * (EPW // WR)
    slab = (slab0, slab1)
    rows = (rows0, rows1)
    gsem = (gsem0, gsem1)
    ssem = (ssem0, ssem1)

    def load(t, b):
        pltpu.sync_copy(eidx_hbm.at[w0 + t], slab[b])

    def fire_g(b):
        pltpu.async_copy(g_sp.at[slab[b].at[0]], rows[b], gsem[b])

    def drain_g(b):
        pltpu.make_async_copy(g_sp.at[slab[b].at[0]], rows[b], gsem[b]).wait()

    def fire_s(b):
        pltpu.async_copy(rows[b], acc_sp.at[slab[b].at[1]], ssem[b], add=True)

    def drain_s(b):
        pltpu.make_async_copy(rows[b], acc_sp.at[slab[b].at[1]],
                              ssem[b]).wait()

    _edge_pipeline(EPW // WR, load, fire_g, drain_g, fire_s, drain_s)

    plsc.subcore_barrier()
    pltpu.sync_copy(acc_sp.at[pl.ds(st, STRIPE)], d0_v)
    pltpu.sync_copy(d0_v, outp_hbm.at[c, pl.ds(st, STRIPE)])


@functools.partial(
    pl.kernel, mesh=_MESH, compiler_params=_PARAMS,
    out_type=jax.ShapeDtypeStruct((NC, NP), F32),
    scratch_types=[
        pltpu.VMEM((2, WR), jnp.int32), pltpu.VMEM((2, WR), jnp.int32),
        pltpu.VMEM((WR,), F32),
        pltpu.VMEM((STRIPE,), F32),
        pltpu.VMEM_SHARED((NP,), F32),
        pltpu.SemaphoreType.DMA, pltpu.SemaphoreType.DMA,
    ])
def _deg_sc(eidx_hbm, zeros_hbm, out_hbm, slab0, slab1, ones_v, stage_v,
            acc_sp, ssem0, ssem1):
    c = lax.axis_index("c")
    s = lax.axis_index("s")
    st = pl.multiple_of(s * STRIPE, 128)
    pltpu.sync_copy(zeros_hbm.at[pl.ds(st, STRIPE)], stage_v)
    pltpu.sync_copy(stage_v, acc_sp.at[pl.ds(st, STRIPE)])
    one = jnp.full((16,), 1.0, F32)

    @pl.loop(0, WR // 16)
    def _(i):
        ones_v[pl.ds(i * 16, 16)] = one

    plsc.subcore_barrier()

    w0 = (c * NS + s) * (EPW // WR)
    slab = (slab0, slab1)
    ssem = (ssem0, ssem1)

    def load(t, b):
        pltpu.sync_copy(eidx_hbm.at[w0 + t], slab[b])

    def fire(b):
        pltpu.async_copy(ones_v, acc_sp.at[slab[b].at[1]], ssem[b], add=True)

    def drain(b):
        pltpu.make_async_copy(ones_v, acc_sp.at[slab[b].at[1]],
                              ssem[b]).wait()

    nwin = EPW // WR
    load(0, 0)
    fire(0)
    load(1, 1)
    fire(1)

    @pl.loop(0, (nwin - 2) // 2)
    def _(i):
        for b in (0, 1):
            drain(b)
            load(2 * i + 2 + b, b)
            fire(b)

    drain(0)
    drain(1)
    plsc.subcore_barrier()
    pltpu.sync_copy(acc_sp.at[pl.ds(st, STRIPE)], stage_v)
    pltpu.sync_copy(stage_v, out_hbm.at[c, pl.ds(st, STRIPE)])


def _make_propF(F, feature_split):
    @functools.partial(
        pl.kernel, mesh=_MESH, compiler_params=_PARAMS,
        out_type=jax.ShapeDtypeStruct((NC, NP, F), F32),
        scratch_types=[
            pltpu.VMEM((2, WR), jnp.int32),
            pltpu.VMEM((WR, F), F32),
            pltpu.VMEM((CH, F), F32),
            pltpu.VMEM_SHARED((NPS, F), F32),
        ])
    def k(eidx_hbm, g_hbm, out_hbm, slab_v, rows_v, stage_v, acc_sp):
        c = lax.axis_index("c")
        s = lax.axis_index("s")
        sts = pl.multiple_of(s * STRIPE_S, 8)

        if feature_split:
            w0 = s * (EPT // WR)
            nwin = EPT // WR
        else:
            w0 = (c * NS + s) * (EPW // WR)
            nwin = EPW // WR

        g_src = g_hbm.at[c] if feature_split else g_hbm

        for j in range(STRIPE_S // CH):
            r = sts + j * CH
            pltpu.sync_copy(g_src.at[pl.ds(r, CH)], stage_v)
            pltpu.sync_copy(stage_v, acc_sp.at[pl.ds(r, CH)])
        plsc.subcore_barrier()

        @pl.loop(0, nwin)
        def _(t):
            pltpu.sync_copy(eidx_hbm.at[w0 + t], slab_v)
            pltpu.sync_copy(g_src.at[slab_v.at[0]], rows_v)
            pltpu.sync_copy(rows_v, acc_sp.at[slab_v.at[1]], add=True)

        plsc.subcore_barrier()
        for j in range(STRIPE_S // CH):
            r = sts + j * CH
            pltpu.sync_copy(acc_sp.at[pl.ds(r, CH)], stage_v)
            pltpu.sync_copy(stage_v, out_hbm.at[c, pl.ds(r, CH)])

    return k


_propf_sc = _make_propF(16, feature_split=False)
_prop32_sc = _make_propF(16, feature_split=True)


BN = 10240
GRID = NP // BN


def _dot_ref(h, w):
    return jnp.dot(h.astype(jnp.bfloat16), w.astype(jnp.bfloat16),
                   preferred_element_type=F32)


def _b2_body(p1_ref, g1_ref, dinv_ref, w1_ref, b1_ref, w2_ref, out_ref):
    dinv = dinv_ref[...]
    p0 = dinv * (p1_ref[0] + p1_ref[1] - g1_ref[...])
    h1 = jnp.maximum(p0[:, None] * w1_ref[0][None, :] + b1_ref[...], 0.0)
    a2 = _dot_ref(h1, w2_ref[...])
    g2 = dinv[:, None] * a2
    out_ref[0] = g2[:, :16]
    out_ref[1] = g2[:, 16:]


def _b3_body(p2_ref, dinv_ref, b2_ref, w3_ref, out_ref):
    dinv = dinv_ref[...]
    acc = jnp.concatenate([p2_ref[0], p2_ref[1]], axis=-1)
    h2 = jnp.maximum(dinv[:, None] * acc + b2_ref[...], 0.0)
    a3 = _dot_ref(h2, w3_ref[...])
    out_ref[...] = dinv[:, None] * a3


def _b4_body(p3_ref, g3_ref, dinv_ref, b3_ref, w4_ref, out_ref):
    dinv = dinv_ref[...]
    acc = p3_ref[0] + p3_ref[1] - g3_ref[...]
    h3 = jnp.maximum(dinv[:, None] * acc + b3_ref[...], 0.0)
    a4 = _dot_ref(h3, jnp.transpose(w4_ref[...]))
    out_ref[...] = dinv * a4[:, 0]


def _b5_body(p4_ref, g4_ref, dinv_ref, b4_ref, out_ref):
    out_ref[...] = (dinv_ref[...] * (p4_ref[0] + p4_ref[1] - g4_ref[...])
                    + b4_ref[0, 0])


def _vec_spec():
    return pl.BlockSpec((BN,), lambda i: (i,))


def _pair_spec():
    return pl.BlockSpec((2, BN), lambda i: (0, i))


def _mat_spec(F):
    return pl.BlockSpec((BN, F), lambda i: (i, 0))


def _pairmat_spec(F):
    return pl.BlockSpec((2, BN, F), lambda i: (0, i, 0))


def _full(shape):
    return pl.BlockSpec(shape, lambda i: tuple(0 for _ in shape))


def kernel(x, edge_index, W1, b1, W2, b2, W3, b3, W4, b4):
    src = edge_index[0].astype(jnp.int32)
    dst = edge_index[1].astype(jnp.int32)

    pad_e = EPAD - E
    pad_ids = (N + (jnp.arange(pad_e, dtype=jnp.int32) % (NPS - N)))
    srcp = jnp.concatenate([src, pad_ids])
    dstp = jnp.concatenate([dst, pad_ids])
    eidx = jnp.stack([srcp.reshape(NSLAB, WR), dstp.reshape(NSLAB, WR)],
                     axis=1)

    xp = jnp.pad(x[:, 0], (0, NP - N))
    zeros_n = jnp.zeros((NP,), F32)
    b1r = b1.reshape(1, 64)
    b2r = b2.reshape(1, 32)
    b3r = b3.reshape(1, 16)
    b4r = b4.reshape(1, 1)

    degp = _deg_sc(eidx, zeros_n)
    p1, dinv, g1 = _prop1x_sc(eidx, degp, xp)

    g2 = pl.pallas_call(
        _b2_body,
        grid=(GRID,),
        in_specs=[_pair_spec(), _vec_spec(), _vec_spec(),
                  _full((1, 64)), _full((1, 64)), _full((64, 32))],
        out_specs=_pairmat_spec(16),
        out_shape=jax.ShapeDtypeStruct((2, NP, 16), F32),
    )(p1, g1, dinv, W1, b1r, W2)

    p2 = _prop32_sc(eidx, g2)

    g3 = pl.pallas_call(
        _b3_body,
        grid=(GRID,),
        in_specs=[_pairmat_spec(16), _vec_spec(),
                  _full((1, 32)), _full((32, 16))],
        out_specs=_mat_spec(16),
        out_shape=jax.ShapeDtypeStruct((NP, 16), F32),
    )(p2, dinv, b2r, W3)

    p3 = _propf_sc(eidx, g3)

    g4 = pl.pallas_call(
        _b4_body,
        grid=(GRID,),
        in_specs=[_pairmat_spec(16), _mat_spec(16), _vec_spec(),
                  _full((1, 16)), _full((1, 16))],
        out_specs=_vec_spec(),
        out_shape=jax.ShapeDtypeStruct((NP,), F32),
    )(p3, g3, dinv, b3r, W4.reshape(1, 16))

    p4 = _prop1_sc(eidx, g4)

    out = pl.pallas_call(
        _b5_body,
        grid=(GRID,),
        in_specs=[_pair_spec(), _vec_spec(), _vec_spec(), _full((1, 1))],
        out_specs=_vec_spec(),
        out_shape=jax.ShapeDtypeStruct((NP,), F32),
    )(p4, g4, dinv, b4r)

    return out[:N].reshape(N, 1)
